# baseline (speedup 1.0000x reference)
import functools

import jax
import jax.numpy as jnp
from jax import lax
from jax.experimental import pallas as pl
from jax.experimental.pallas import tpu as pltpu
from jax.experimental.pallas import tpu_sc as plsc

_TOP_K = 2
_E = 64
_H = 2048


def _matmul_body(chunk, x_ref, w_ref, out_ref):
    res = lax.dot_general(
        w_ref[...],
        x_ref[...],
        (((1,), (1,)), ((), ())),
        preferred_element_type=jnp.float32,
    )
    cpb, nq = out_ref.shape[0], out_ref.shape[1]
    sub = chunk // nq
    for c in range(cpb):
        for q in range(nq):
            off = c * chunk + q * sub
            out_ref[c, q] = res[:, off:off + sub]


def _logits_blocked(x, w, bm, chunk, nw, nq):
    rows = x.shape[0]
    cpb = bm // chunk
    sub = chunk // nq
    return pl.pallas_call(
        functools.partial(_matmul_body, chunk),
        grid=(rows // bm,),
        in_specs=[
            pl.BlockSpec((bm, _H), lambda i: (i, 0)),
            pl.BlockSpec((_E, _H), lambda i: (0, 0)),
        ],
        out_specs=pl.BlockSpec((cpb, nq, _E, sub), lambda i: (i, 0, 0, 0)),
        out_shape=jax.ShapeDtypeStruct((nw, nq, _E, sub), jnp.float32),
    )(x, w)


def _make_router(t, nq):
    info = plsc.get_sparse_core_info()
    nc, ns, lanes = info.num_cores, info.num_subcores, info.num_lanes
    nw = nc * ns
    chunk = t // nw
    sub = chunk // nq
    gq = sub // lanes
    mesh = plsc.VectorSubcoreMesh(core_axis_name="c", subcore_axis_name="s")

    @functools.partial(
        pl.kernel,
        out_type=(
            jax.ShapeDtypeStruct((_TOP_K, t), jnp.int32),
            jax.ShapeDtypeStruct((_TOP_K, t), jnp.float32),
        ),
        mesh=mesh,
        scratch_types=[
            pltpu.VMEM((nq, _E, sub), jnp.float32),
            pltpu.VMEM((_TOP_K, chunk), jnp.int32),
            pltpu.VMEM((_TOP_K, chunk), jnp.float32),
            pltpu.SemaphoreType.DMA((nq,)),
        ],
    )
    def router(logits_hbm, idx_hbm, w_hbm, buf, idx_v, w_v, sems):
        wid = lax.axis_index("s") * nc + lax.axis_index("c")
        base = wid * chunk
        copies = [
            pltpu.async_copy(logits_hbm.at[wid, q], buf.at[q], sems.at[q])
            for q in range(nq)
        ]

        def make_group(q):
            def group(g, carry):
                neg = jnp.full((lanes,), -jnp.inf, jnp.float32)
                zero_i = jnp.zeros((lanes,), jnp.int32)

                def expert(e, c):
                    m1, i1, m2, i2 = c
                    v = buf[q, e, pl.ds(g * lanes, lanes)]
                    e_vec = jnp.broadcast_to(e, (lanes,)).astype(jnp.int32)
                    gt1 = v > m1
                    gt2 = v > m2
                    m2n = jnp.where(gt1, m1, jnp.where(gt2, v, m2))
                    i2n = jnp.where(gt1, i1, jnp.where(gt2, e_vec, i2))
                    m1n = jnp.where(gt1, v, m1)
                    i1n = jnp.where(gt1, e_vec, i1)
                    return m1n, i1n, m2n, i2n

                m1, i1, m2, i2 = lax.fori_loop(
                    0, _E, expert, (neg, zero_i, neg, zero_i), unroll=16
                )
                d = jnp.exp(m2 - m1)
                w1 = 1.0 / (1.0 + d)
                w2 = 1.0 - w1
                sl = pl.ds(q * sub + g * lanes, lanes)
                idx_v[0, sl] = i1
                idx_v[1, sl] = i2
                w_v[0, sl] = w1
                w_v[1, sl] = w2
                return carry

            return group

        for q in range(nq):
            copies[q].wait()
            lax.fori_loop(0, gq, make_group(q), 0)
        pltpu.sync_copy(idx_v, idx_hbm.at[:, pl.ds(base, chunk)])
        pltpu.sync_copy(w_v, w_hbm.at[:, pl.ds(base, chunk)])

    return router


def kernel(hidden_states, weight):
    bsz, seq_len, h = hidden_states.shape
    x = hidden_states.reshape(-1, h)
    t = x.shape[0]
    info = plsc.get_sparse_core_info()
    nw = info.num_cores * info.num_subcores
    nq = 4
    logits_b = _logits_blocked(x, weight, 1024, t // nw, nw, nq)
    idx, wts = _make_router(t, nq)(logits_b)
    return (idx.T.reshape(bsz, seq_len, _TOP_K),
            wts.T.reshape(bsz, seq_len, _TOP_K))


# speedup vs baseline: 1.0109x; 1.0109x over previous
import functools

import jax
import jax.numpy as jnp
from jax import lax
from jax.experimental import pallas as pl
from jax.experimental.pallas import tpu as pltpu
from jax.experimental.pallas import tpu_sc as plsc

_TOP_K = 2
_E = 64
_H = 2048


def _matmul_body(chunk, x_ref, w_ref, out_ref):
    res = lax.dot_general(
        w_ref[...],
        x_ref[...],
        (((1,), (1,)), ((), ())),
        preferred_element_type=jnp.float32,
    )
    cpb = out_ref.shape[0]
    for c in range(cpb):
        out_ref[c] = res[:, c * chunk:(c + 1) * chunk]


def _logits_blocked(x, w, bm, chunk, nw):
    rows = x.shape[0]
    cpb = bm // chunk
    return pl.pallas_call(
        functools.partial(_matmul_body, chunk),
        grid=(rows // bm,),
        in_specs=[
            pl.BlockSpec((bm, _H), lambda i: (i, 0)),
            pl.BlockSpec((_E, _H), lambda i: (0, 0)),
        ],
        out_specs=pl.BlockSpec((cpb, _E, chunk), lambda i: (i, 0, 0)),
        out_shape=jax.ShapeDtypeStruct((nw, _E, chunk), jnp.float32),
    )(x, w)


def _make_router(t):
    info = plsc.get_sparse_core_info()
    nc, ns, lanes = info.num_cores, info.num_subcores, info.num_lanes
    nw = nc * ns
    chunk = t // nw
    ngroups = chunk // lanes
    mesh = plsc.VectorSubcoreMesh(core_axis_name="c", subcore_axis_name="s")

    @functools.partial(
        pl.kernel,
        out_type=(
            jax.ShapeDtypeStruct((_TOP_K, t), jnp.int32),
            jax.ShapeDtypeStruct((_TOP_K, t), jnp.float32),
        ),
        mesh=mesh,
        scratch_types=[
            pltpu.VMEM((_E, chunk), jnp.float32),
            pltpu.VMEM((_TOP_K, chunk), jnp.int32),
            pltpu.VMEM((_TOP_K, chunk), jnp.float32),
        ],
    )
    def router(logits_hbm, idx_hbm, w_hbm, buf, idx_v, w_v):
        wid = lax.axis_index("s") * nc + lax.axis_index("c")
        base = wid * chunk
        pltpu.sync_copy(logits_hbm.at[wid], buf)

        def group(g, carry):
            neg = jnp.full((lanes,), -jnp.inf, jnp.float32)
            zero_i = jnp.zeros((lanes,), jnp.int32)

            def expert(e, c):
                m1, i1, m2, i2 = c
                v = buf[e, pl.ds(g * lanes, lanes)]
                e_vec = jnp.broadcast_to(e, (lanes,)).astype(jnp.int32)
                gt1 = v > m1
                gt2 = v > m2
                m2n = jnp.where(gt1, m1, jnp.where(gt2, v, m2))
                i2n = jnp.where(gt1, i1, jnp.where(gt2, e_vec, i2))
                m1n = jnp.where(gt1, v, m1)
                i1n = jnp.where(gt1, e_vec, i1)
                return m1n, i1n, m2n, i2n

            m1, i1, m2, i2 = lax.fori_loop(
                0, _E, expert, (neg, zero_i, neg, zero_i), unroll=8
            )
            d = jnp.exp(m2 - m1)
            w1 = 1.0 / (1.0 + d)
            w2 = 1.0 - w1
            sl = pl.ds(g * lanes, lanes)
            idx_v[0, sl] = i1
            idx_v[1, sl] = i2
            w_v[0, sl] = w1
            w_v[1, sl] = w2
            return carry

        lax.fori_loop(0, ngroups, group, 0)
        pltpu.sync_copy(idx_v, idx_hbm.at[:, pl.ds(base, chunk)])
        pltpu.sync_copy(w_v, w_hbm.at[:, pl.ds(base, chunk)])

    return router


def kernel(hidden_states, weight):
    bsz, seq_len, h = hidden_states.shape
    x = hidden_states.reshape(-1, h)
    t = x.shape[0]
    info = plsc.get_sparse_core_info()
    nw = info.num_cores * info.num_subcores
    logits_b = _logits_blocked(x, weight, 1024, t // nw, nw)
    idx, wts = _make_router(t)(logits_b)
    return (idx.T.reshape(bsz, seq_len, _TOP_K),
            wts.T.reshape(bsz, seq_len, _TOP_K))
